# baseline (speedup 1.0000x reference)
import jax
import jax.numpy as jnp
from jax import lax
from jax.experimental import pallas as pl
from jax.experimental.pallas import tpu as pltpu

D = 4096
E = 64
T = 8192
BT = 1024
BK = 512
KB = D // BK


def _body(x_ref, w_ref, oh_ref, tp_ref, lg_ref, acc_ref):
    k = pl.program_id(1)
    part = lax.dot_general(
        x_ref[...], w_ref[:, pl.ds(k * BK, BK)],
        (((1,), (1,)), ((), ())),
        preferred_element_type=jnp.float32,
    )

    @pl.when(k == 0)
    def _():
        acc_ref[...] = part

    @pl.when(k > 0)
    def _():
        acc_ref[...] += part

    @pl.when(k == KB - 1)
    def _():
        lg = acc_ref[...]
        lg_ref[...] = lg
        m = jnp.max(lg, axis=1, keepdims=True)
        iota = lax.broadcasted_iota(jnp.int32, (BT, E), 1)
        am = jnp.min(jnp.where(lg == m, iota, E), axis=1, keepdims=True)
        oh_ref[...] = (iota == am).astype(jnp.int32)
        tp_ref[...] = 1.0 / jnp.sum(jnp.exp(lg - m), axis=1, keepdims=True)


def kernel(x, W):
    oh, tp, lg = pl.pallas_call(
        _body,
        grid=(T // BT, KB),
        in_specs=[
            pl.BlockSpec((BT, BK), lambda i, k: (i, k)),
            pl.BlockSpec((E, D), lambda i, k: (0, 0)),
        ],
        out_specs=(
            pl.BlockSpec((BT, E), lambda i, k: (i, 0)),
            pl.BlockSpec((BT, 1), lambda i, k: (i, 0)),
            pl.BlockSpec((BT, E), lambda i, k: (i, 0)),
        ),
        out_shape=(
            jax.ShapeDtypeStruct((T, E), jnp.int32),
            jax.ShapeDtypeStruct((T, 1), jnp.float32),
            jax.ShapeDtypeStruct((T, E), jnp.float32),
        ),
        scratch_shapes=[pltpu.VMEM((BT, E), jnp.float32)],
        compiler_params=pltpu.CompilerParams(
            dimension_semantics=("arbitrary", "arbitrary"),
        ),
    )(x, W)
    return oh, tp, lg


# speedup vs baseline: 1.5264x; 1.5264x over previous
import jax
import jax.numpy as jnp
from jax import lax
from jax.experimental import pallas as pl
from jax.experimental.pallas import tpu as pltpu

D = 4096
E = 64
T = 8192
CT = 1024
NCHUNK = T // CT
NBUF = 3

_tc_mesh = pltpu.create_tensorcore_mesh("t", num_cores=1)


def _x_copy(x_hbm, xbufs, xsems, i, b):
    return pltpu.make_async_copy(
        x_hbm.at[pl.ds(i * CT, CT), :], xbufs.at[b], xsems.at[b])


def _tc_body(x_hbm, w_hbm, oh_hbm, tp_hbm, lg_hbm,
             wbuf, xbufs, ohbufs, tpbufs, lgbufs,
             wsem, xsems, osems):
    pltpu.async_copy(w_hbm, wbuf, wsem).wait()
    for b in range(NBUF):
        _x_copy(x_hbm, xbufs, xsems, b, b).start()
    w = wbuf[...]
    iota = lax.broadcasted_iota(jnp.int32, (CT, E), 1)
    for i in range(NCHUNK):
        b = i % NBUF
        _x_copy(x_hbm, xbufs, xsems, i, b).wait()
        lg = lax.dot_general(
            xbufs[b], w,
            (((1,), (1,)), ((), ())),
            preferred_element_type=jnp.float32,
        )
        nxt = i + NBUF
        if nxt < NCHUNK:
            _x_copy(x_hbm, xbufs, xsems, nxt, b).start()
        m = jnp.max(lg, axis=1, keepdims=True)
        am = jnp.min(jnp.where(lg == m, iota, E), axis=1, keepdims=True)
        oh = (iota == am).astype(jnp.int32)
        tp = 1.0 / jnp.sum(jnp.exp(lg - m), axis=1, keepdims=True)
        ob = i % 2
        if i >= 2:
            j = i - 2
            pltpu.make_async_copy(
                ohbufs.at[ob], oh_hbm.at[pl.ds(j * CT, CT), :],
                osems.at[ob, 0]).wait()
            pltpu.make_async_copy(
                tpbufs.at[ob], tp_hbm.at[pl.ds(j * CT, CT), :],
                osems.at[ob, 1]).wait()
            pltpu.make_async_copy(
                lgbufs.at[ob], lg_hbm.at[pl.ds(j * CT, CT), :],
                osems.at[ob, 2]).wait()
        ohbufs[ob] = oh
        tpbufs[ob] = tp
        lgbufs[ob] = lg
        pltpu.make_async_copy(
            ohbufs.at[ob], oh_hbm.at[pl.ds(i * CT, CT), :],
            osems.at[ob, 0]).start()
        pltpu.make_async_copy(
            tpbufs.at[ob], tp_hbm.at[pl.ds(i * CT, CT), :],
            osems.at[ob, 1]).start()
        pltpu.make_async_copy(
            lgbufs.at[ob], lg_hbm.at[pl.ds(i * CT, CT), :],
            osems.at[ob, 2]).start()
    for i in (NCHUNK - 2, NCHUNK - 1):
        ob = i % 2
        pltpu.make_async_copy(
            ohbufs.at[ob], oh_hbm.at[pl.ds(i * CT, CT), :],
            osems.at[ob, 0]).wait()
        pltpu.make_async_copy(
            tpbufs.at[ob], tp_hbm.at[pl.ds(i * CT, CT), :],
            osems.at[ob, 1]).wait()
        pltpu.make_async_copy(
            lgbufs.at[ob], lg_hbm.at[pl.ds(i * CT, CT), :],
            osems.at[ob, 2]).wait()


_router = pl.kernel(
    _tc_body,
    out_type=(
        jax.ShapeDtypeStruct((T, E), jnp.int32),
        jax.ShapeDtypeStruct((T, 1), jnp.float32),
        jax.ShapeDtypeStruct((T, E), jnp.float32),
    ),
    mesh=_tc_mesh,
    scratch_types=[
        pltpu.VMEM((E, D), jnp.float32),
        pltpu.VMEM((NBUF, CT, D), jnp.float32),
        pltpu.VMEM((2, CT, E), jnp.int32),
        pltpu.VMEM((2, CT, 1), jnp.float32),
        pltpu.VMEM((2, CT, E), jnp.float32),
        pltpu.SemaphoreType.DMA,
        pltpu.SemaphoreType.DMA((NBUF,)),
        pltpu.SemaphoreType.DMA((2, 3)),
    ],
)


def kernel(x, W):
    oh, tp, lg = _router(x, W)
    return oh, tp, lg


# speedup vs baseline: 1.6057x; 1.0519x over previous
import jax
import jax.numpy as jnp
from jax import lax
from jax.experimental import pallas as pl
from jax.experimental.pallas import tpu as pltpu

D = 4096
E = 64
T = 8192
BT = 1024


def _body(x_ref, w_ref, oh_ref, tp_ref, lg_ref):
    lg = lax.dot_general(
        x_ref[...], w_ref[...],
        (((1,), (1,)), ((), ())),
        preferred_element_type=jnp.float32,
    )
    lg_ref[...] = lg
    m = jnp.max(lg, axis=1, keepdims=True)
    iota = lax.broadcasted_iota(jnp.int32, (BT, E), 1)
    am = jnp.min(jnp.where(lg == m, iota, E), axis=1, keepdims=True)
    oh_ref[...] = (iota == am).astype(jnp.int32)
    tp_ref[...] = 1.0 / jnp.sum(jnp.exp(lg - m), axis=1, keepdims=True)


def kernel(x, W):
    oh, tp, lg = pl.pallas_call(
        _body,
        grid=(T // BT,),
        in_specs=[
            pl.BlockSpec((BT, D), lambda i: (i, 0)),
            pl.BlockSpec((E, D), lambda i: (0, 0)),
        ],
        out_specs=(
            pl.BlockSpec((BT, E), lambda i: (i, 0)),
            pl.BlockSpec((BT, 1), lambda i: (i, 0)),
            pl.BlockSpec((BT, E), lambda i: (i, 0)),
        ),
        out_shape=(
            jax.ShapeDtypeStruct((T, E), jnp.int32),
            jax.ShapeDtypeStruct((T, 1), jnp.float32),
            jax.ShapeDtypeStruct((T, E), jnp.float32),
        ),
        compiler_params=pltpu.CompilerParams(
            dimension_semantics=("arbitrary",),
        ),
    )(x, W)
    return oh, tp, lg
